# baseline (speedup 1.0000x reference)
import functools

import jax
import jax.numpy as jnp
from jax import lax
from jax.experimental import pallas as pl
from jax.experimental.pallas import tpu as pltpu
from jax.experimental.pallas import tpu_sc as plsc

NC = 2
NS = 16
L = 16
NW = NC * NS

R, C, F = 4096, 2048, 8
SLABS = R // 8
SLABS_PW = SLABS // NW
QW = 256
NQ = C // QW
NCHUNK = SLABS_PW * NQ
NBUF = 4
NGRP = NCHUNK // NBUF


def _sc_body(x_hbm, et_hbm, out_hbm, *scratch):
    xvs = scratch[0:NBUF]
    ovs = scratch[NBUF:2 * NBUF]
    etv = scratch[2 * NBUF]
    lds = scratch[2 * NBUF + 1:2 * NBUF + 1 + NBUF]
    sts = scratch[2 * NBUF + 1 + NBUF:2 * NBUF + 1 + 2 * NBUF]

    wid = lax.axis_index("s") * NC + lax.axis_index("c")
    slab0 = wid * SLABS_PW
    pltpu.sync_copy(et_hbm, etv)
    ev = etv[pl.ds(0, L)]
    e0b = [jnp.broadcast_to(ev[f], (L,)) for f in range(F)]
    e1b = [jnp.broadcast_to(ev[F + f], (L,)) for f in range(F)]

    def addr(i):
        r0 = (slab0 + i // NQ) * 8
        q = (i % NQ) * QW
        return r0, q

    def load(i, xv, sem):
        r0, q = addr(i)
        return pltpu.make_async_copy(
            x_hbm.at[pl.ds(r0, 8), pl.ds(q, QW)], xv, sem)

    def store(i, ov, sem):
        r0, q = addr(i)
        return pltpu.make_async_copy(
            ov, out_hbm.at[pl.ds(r0, 8), :, pl.ds(q, QW)], sem)

    def compute(xv, ov):
        @plsc.parallel_loop(0, 8 * (QW // L), 1, unroll=2)
        def inner(it):
            s = it // (QW // L)
            v = (it % (QW // L)) * L
            m = xv[s, pl.ds(v, L)] > 0
            for f in range(F):
                ov[s, f, pl.ds(v, L)] = jnp.where(m, e1b[f], e0b[f])

    bufs = tuple(zip(xvs, ovs, lds, sts))

    for b in range(NBUF):
        load(b, xvs[b], lds[b]).start()
    for b, (xv, ov, ld, st) in enumerate(bufs):
        load(b, xv, ld).wait()
        compute(xv, ov)
        store(b, ov, st).start()
        load(b + NBUF, xv, ld).start()

    def grp_body(g, carry):
        for b, (xv, ov, ld, st) in enumerate(bufs):
            i = NBUF * g + b
            load(i, xv, ld).wait()
            store(i - NBUF, ov, st).wait()
            compute(xv, ov)
            store(i, ov, st).start()
            load(i + NBUF, xv, ld).start()
        return carry

    lax.fori_loop(1, NGRP - 1, grp_body, 0)

    for b, (xv, ov, ld, st) in enumerate(bufs):
        i = NCHUNK - NBUF + b
        load(i, xv, ld).wait()
        store(i - NBUF, ov, st).wait()
        compute(xv, ov)
        store(i, ov, st).start()
    for b, (xv, ov, ld, st) in enumerate(bufs):
        store(NCHUNK - NBUF + b, ov, st).wait()


@jax.jit
def kernel(x, embedding):
    et = embedding.reshape(-1)
    run = functools.partial(
        pl.kernel,
        out_type=jax.ShapeDtypeStruct((R, F, C), jnp.float32),
        mesh=plsc.VectorSubcoreMesh(core_axis_name="c", subcore_axis_name="s"),
        compiler_params=pltpu.CompilerParams(use_tc_tiling_on_sc=True),
        scratch_types=(
            [pltpu.VMEM((8, QW), jnp.float32) for _ in range(NBUF)]
            + [pltpu.VMEM((8, F, QW), jnp.float32) for _ in range(NBUF)]
            + [pltpu.VMEM((2 * F,), jnp.float32)]
            + [pltpu.SemaphoreType.DMA for _ in range(2 * NBUF)]
        ),
    )(_sc_body)
    z = run(x, et)
    return z.transpose(0, 2, 1)
